# baseline (speedup 1.0000x reference)
import functools

import jax
import jax.numpy as jnp
from jax import lax
from jax.experimental import pallas as pl
from jax.experimental.pallas import tpu as pltpu
from jax.experimental.pallas import tpu_sc as plsc

B = 4096
NG = 18
YEAR_LO, YEAR_HI = 1919, 2000
YSPAN = YEAR_HI - YEAR_LO + 1
D_ITEM = 64


VOCAB = 100000
KIN = 1024
NPAIR = ((VOCAB + KIN - 1) // KIN) * (KIN // 2)


def _tc_transpose(item_table_t):
    def body(in_ref, out_ref):
        g = pl.program_id(0)
        lane = g * KIN + lax.broadcasted_iota(jnp.int32, (D_ITEM, KIN), 1)
        x = jnp.where(lane < VOCAB, in_ref[...], 0.0)
        t = jnp.swapaxes(x, 0, 1)
        for k in range(KIN // 256):
            out_ref[k * 128:(k + 1) * 128, 0:D_ITEM] = t[2 * k * 128:(2 * k + 1) * 128, :]
            out_ref[k * 128:(k + 1) * 128, D_ITEM:] = t[(2 * k + 1) * 128:(2 * k + 2) * 128, :]

    return pl.pallas_call(
        body,
        grid=(NPAIR * 2 // KIN,),
        in_specs=[pl.BlockSpec((D_ITEM, KIN), lambda g: (0, g))],
        out_specs=pl.BlockSpec((KIN // 2, 2 * D_ITEM), lambda g: (g, 0)),
        out_shape=jax.ShapeDtypeStruct((NPAIR, 2 * D_ITEM), jnp.float32),
    )(item_table_t)


def _sc_gather(item_ids, itab2):
    info = plsc.get_sparse_core_info()
    nw = info.num_cores * info.num_subcores
    bpw = B // nw

    mesh = plsc.VectorSubcoreMesh(core_axis_name="c", subcore_axis_name="s")

    @functools.partial(
        pl.kernel,
        mesh=mesh,
        compiler_params=pltpu.CompilerParams(use_tc_tiling_on_sc=True),
        out_type=jax.ShapeDtypeStruct((B, 2 * D_ITEM), jnp.float32),
        scratch_types=[
            pltpu.VMEM((bpw,), jnp.int32),
            pltpu.VMEM((bpw, 2 * D_ITEM), jnp.float32),
            pltpu.SemaphoreType.DMA,
        ],
    )
    def gather_kernel(ids_hbm, itab_hbm, out_hbm, idx_v, staged_v, sem):
        wid = lax.axis_index("s") * info.num_cores + lax.axis_index("c")
        base = wid * bpw
        pltpu.sync_copy(ids_hbm.at[pl.ds(base, bpw)], idx_v)
        for i in range(bpw // 16):
            v = idx_v[pl.ds(i * 16, 16)]
            j = (jax.lax.shift_right_logical(v, 8) * 128
                 + jnp.bitwise_and(v, 127))
            idx_v[pl.ds(i * 16, 16)] = j
        pltpu.async_copy(itab_hbm.at[idx_v], staged_v, sem).wait()
        pltpu.sync_copy(staged_v, out_hbm.at[pl.ds(base, bpw)])

    return gather_kernel(item_ids, itab2)


def _ln(x, g, b):
    m = jnp.mean(x, axis=-1, keepdims=True)
    v = jnp.mean((x - m) * (x - m), axis=-1, keepdims=True)
    return (x - m) * lax.rsqrt(v + 1e-5) * g + b


def _dot(a, b):
    return jnp.dot(a, b, preferred_element_type=jnp.float32)


def _dot_t(a, bt):
    return lax.dot_general(a, bt, (((1,), (1,)), ((), ())),
                           preferred_element_type=jnp.float32)


def _tc_body(slab_ref, ids_ref, yrs_ref, gv_ref, title_ref,
             ytab_ref,
             wg_ref, bg_ref, wt1_ref, bt1_ref, wt2_ref, bt2_ref,
             wb0_ref, bb0_ref, g0_ref, be0_ref,
             wb1_ref, bb1_ref, g1_ref, be1_ref,
             wb2_ref, bb2_ref, g2_ref, be2_ref,
             wattn_ref, battn_ref,
             r1_ref, rb1_ref, r2_ref, rb2_ref, r3_ref, rb3_ref,
             wagg_ref, bagg_ref, wo_ref, bo_ref, go_ref, beo_ref,
             out_ref):
    m = jnp.bitwise_and(jax.lax.shift_right_logical(ids_ref[...], 7),
                        1).astype(jnp.float32)
    item_emb = slab_ref[:, 0:D_ITEM] * (1.0 - m) + slab_ref[:, D_ITEM:2 * D_ITEM] * m

    yi = jnp.clip(yrs_ref[...] - YEAR_LO, 0, YSPAN - 1)
    onehot = (lax.broadcasted_iota(jnp.int32, (yi.shape[0], YSPAN + 1), 1)
              == yi).astype(jnp.float32)
    year_emb = _dot_t(onehot, ytab_ref[...])

    gvf = gv_ref[...].astype(jnp.float32)
    genre_emb = jax.nn.relu(_dot(gvf, wg_ref[...]) + bg_ref[...])
    t = jax.nn.relu(_dot_t(title_ref[...], wt1_ref[...]) + bt1_ref[...])
    text_emb = _dot(t, wt2_ref[...]) + bt2_ref[...]

    x = (_dot(item_emb, wb0_ref[0:64, :])
         + _dot(genre_emb, wb0_ref[64:96, :])
         + _dot(year_emb, wb0_ref[96:112, :])
         + _dot(text_emb, wb0_ref[112:208, :])
         + bb0_ref[...])
    x = _ln(jax.nn.relu(x), g0_ref[...], be0_ref[...])
    x = _ln(jax.nn.relu(_dot(x, wb1_ref[...]) + bb1_ref[...]), g1_ref[...], be1_ref[...])
    x = _ln(jax.nn.relu(_dot(x, wb2_ref[...]) + bb2_ref[...]), g2_ref[...], be2_ref[...])

    logits = _dot_t(x, wattn_ref[...]) + battn_ref[...]
    z = logits - jnp.max(logits, axis=-1, keepdims=True)
    e = jnp.exp(z)
    gw = e / jnp.sum(e, axis=-1, keepdims=True)
    w = gw * gvf * (gvf > 0.0).astype(jnp.float32)

    r1cat = jnp.concatenate([r1_ref[g] for g in range(NG)], axis=0)
    rb1cat = jnp.concatenate([rb1_ref[g:g + 1, :] for g in range(NG)], axis=1)
    h1 = jax.nn.relu(_dot_t(x, r1cat) + rb1cat)

    h2s = []
    for g in range(NG):
        h1g = h1[:, g * 64:(g + 1) * 64]
        h2s.append(jax.nn.relu(_dot_t(h1g, r2_ref[g]) + rb2_ref[g:g + 1, :]))
    h2 = jnp.concatenate(h2s, axis=1)
    lane = lax.broadcasted_iota(jnp.int32, (NG, NG * 32), 1)
    row = lax.broadcasted_iota(jnp.int32, (NG, NG * 32), 0)
    expand = (lane // 32 == row).astype(jnp.float32)
    wexp = _dot(w, expand)
    r3cat = jnp.concatenate([r3_ref[g] for g in range(NG)], axis=0)
    refin = _dot(h2 * wexp, r3cat) + _dot(w, rb3_ref[...])

    refined = jax.nn.relu(_dot(x, wagg_ref[0:128, :]) + _dot(refin, wagg_ref[128:160, :])
                          + bagg_ref[...])
    out = _ln(jax.nn.relu(_dot(refined, wo_ref[...]) + bo_ref[...]),
              go_ref[...], beo_ref[...])
    out_ref[...] = out


def _tc_specs(bt):
    def data(d):
        return pl.BlockSpec((bt, d), lambda i: (i, 0))

    def w1(n):
        return pl.BlockSpec((n,), lambda i: (0,))

    def w2(s):
        return pl.BlockSpec(s, lambda i: (0, 0))

    def w3(s):
        return pl.BlockSpec(s, lambda i: (0, 0, 0))

    in_specs = [
        pl.BlockSpec((bt, 2 * D_ITEM), lambda i: (i, 0)),
        data(1), data(1), data(NG), data(384),
        w2((16, YSPAN + 1)),
        w2((NG, 32)), w1(32),
        w2((192, 384)), w1(192),
        w2((192, 96)), w1(96),
        w2((208, 384)), w1(384), w1(384), w1(384),
        w2((384, 256)), w1(256), w1(256), w1(256),
        w2((256, 128)), w1(128), w1(128), w1(128),
        w2((NG, 128)), w1(NG),
        w3((NG, 64, 128)), w2((NG, 64)),
        w3((NG, 32, 64)), w2((NG, 32)),
        w3((NG, 32, 32)), w2((NG, 32)),
        w2((160, 128)), w1(128),
        w2((128, 128)), w1(128), w1(128), w1(128),
    ]
    out_spec = pl.BlockSpec((bt, 128), lambda i: (i, 0))
    return in_specs, out_spec


def _tc_args(slabs, item_ids, release_years, genre_vectors, title_embeddings, p):
    return (
        slabs, item_ids.reshape(B, 1), release_years.reshape(B, 1),
        genre_vectors, title_embeddings,
        p['year_table'].T,
        p['Wg'], p['bg'], p['Wt1'].T, p['bt1'], p['Wt2'], p['bt2'],
        p['Wb0'], p['bb0'], p['g0'], p['be0'],
        p['Wb1'], p['bb1'], p['g1'], p['be1'],
        p['Wb2'], p['bb2'], p['g2'], p['be2'],
        p['Wattn'].T, p['battn'],
        jnp.swapaxes(p['R1'], 1, 2), p['Rb1'],
        jnp.swapaxes(p['R2'], 1, 2), p['Rb2'], p['R3'], p['Rb3'],
        p['Wagg'], p['bagg'],
        p['Wo'], p['bo'], p['go'], p['beo'],
    )


def _tc_forward(slabs, item_ids, release_years, genre_vectors,
                title_embeddings, p, bt=1024):
    in_specs, out_spec = _tc_specs(bt)
    return pl.pallas_call(
        _tc_body,
        grid=(B // bt,),
        in_specs=in_specs,
        out_specs=out_spec,
        out_shape=jax.ShapeDtypeStruct((B, 128), jnp.float32),
    )(*_tc_args(slabs, item_ids, release_years, genre_vectors,
                title_embeddings, p))


def kernel(item_ids, genre_vectors, release_years, title_embeddings, params):
    itab2 = _tc_transpose(params['item_table'].T)
    slabs = _sc_gather(item_ids, itab2)
    return _tc_forward(slabs, item_ids, release_years, genre_vectors,
                       title_embeddings, params)


# speedup vs baseline: 1.3442x; 1.3442x over previous
import functools

import jax
import jax.numpy as jnp
from jax import lax
from jax.experimental import pallas as pl
from jax.experimental.pallas import tpu as pltpu
from jax.experimental.pallas import tpu_sc as plsc

B = 4096
NG = 18
YEAR_LO, YEAR_HI = 1919, 2000
YSPAN = YEAR_HI - YEAR_LO + 1
D_ITEM = 64


VOCAB = 100000
KIN = 4096
NPAIR = ((VOCAB + KIN - 1) // KIN) * (KIN // 2)


def _tc_transpose(item_table_t):
    def body(in_ref, out_ref):
        g = pl.program_id(0)
        lane = g * KIN + lax.broadcasted_iota(jnp.int32, (D_ITEM, KIN), 1)
        x = jnp.where(lane < VOCAB, in_ref[...], 0.0)
        eye = (lax.broadcasted_iota(jnp.int32, (D_ITEM, D_ITEM), 0)
               == lax.broadcasted_iota(jnp.int32, (D_ITEM, D_ITEM), 1)
               ).astype(jnp.float32)
        t = lax.dot_general(x, eye, (((0,), (0,)), ((), ())),
                            preferred_element_type=jnp.float32)
        for k in range(KIN // 256):
            out_ref[k * 128:(k + 1) * 128, 0:D_ITEM] = t[2 * k * 128:(2 * k + 1) * 128, :]
            out_ref[k * 128:(k + 1) * 128, D_ITEM:] = t[(2 * k + 1) * 128:(2 * k + 2) * 128, :]

    return pl.pallas_call(
        body,
        grid=(NPAIR * 2 // KIN,),
        in_specs=[pl.BlockSpec((D_ITEM, KIN), lambda g: (0, g))],
        out_specs=pl.BlockSpec((KIN // 2, 2 * D_ITEM), lambda g: (g, 0)),
        out_shape=jax.ShapeDtypeStruct((NPAIR, 2 * D_ITEM), jnp.float32),
    )(item_table_t)


def _sc_gather(item_ids, itab2):
    info = plsc.get_sparse_core_info()
    nw = info.num_cores * info.num_subcores
    bpw = B // nw

    mesh = plsc.VectorSubcoreMesh(core_axis_name="c", subcore_axis_name="s")

    @functools.partial(
        pl.kernel,
        mesh=mesh,
        compiler_params=pltpu.CompilerParams(use_tc_tiling_on_sc=True),
        out_type=jax.ShapeDtypeStruct((B, 2 * D_ITEM), jnp.float32),
        scratch_types=[
            pltpu.VMEM((bpw,), jnp.int32),
            pltpu.VMEM((bpw, 2 * D_ITEM), jnp.float32),
            pltpu.SemaphoreType.DMA,
        ],
    )
    def gather_kernel(ids_hbm, itab_hbm, out_hbm, idx_v, staged_v, sem):
        wid = lax.axis_index("s") * info.num_cores + lax.axis_index("c")
        base = wid * bpw
        pltpu.sync_copy(ids_hbm.at[pl.ds(base, bpw)], idx_v)
        for i in range(bpw // 16):
            v = idx_v[pl.ds(i * 16, 16)]
            j = (jax.lax.shift_right_logical(v, 8) * 128
                 + jnp.bitwise_and(v, 127))
            idx_v[pl.ds(i * 16, 16)] = j
        pltpu.async_copy(itab_hbm.at[idx_v], staged_v, sem).wait()
        pltpu.sync_copy(staged_v, out_hbm.at[pl.ds(base, bpw)])

    return gather_kernel(item_ids, itab2)


def _ln(x, g, b):
    m = jnp.mean(x, axis=-1, keepdims=True)
    v = jnp.mean((x - m) * (x - m), axis=-1, keepdims=True)
    return (x - m) * lax.rsqrt(v + 1e-5) * g + b


def _dot(a, b):
    return jnp.dot(a, b, preferred_element_type=jnp.float32)


def _dot_t(a, bt):
    return lax.dot_general(a, bt, (((1,), (1,)), ((), ())),
                           preferred_element_type=jnp.float32)


def _tc_body(slab_ref, ids_ref, yrs_ref, gv_ref, title_ref,
             ytab_ref,
             wg_ref, bg_ref, wt1_ref, bt1_ref, wt2_ref, bt2_ref,
             wb0_ref, bb0_ref, g0_ref, be0_ref,
             wb1_ref, bb1_ref, g1_ref, be1_ref,
             wb2_ref, bb2_ref, g2_ref, be2_ref,
             wattn_ref, battn_ref,
             r1_ref, rb1_ref, r2_ref, rb2_ref, r3_ref, rb3_ref,
             wagg_ref, bagg_ref, wo_ref, bo_ref, go_ref, beo_ref,
             out_ref):
    m = jnp.bitwise_and(jax.lax.shift_right_logical(ids_ref[...], 7),
                        1).astype(jnp.float32)
    item_emb = slab_ref[:, 0:D_ITEM] * (1.0 - m) + slab_ref[:, D_ITEM:2 * D_ITEM] * m

    yi = jnp.clip(yrs_ref[...] - YEAR_LO, 0, YSPAN - 1)
    onehot = (lax.broadcasted_iota(jnp.int32, (yi.shape[0], YSPAN + 1), 1)
              == yi).astype(jnp.float32)
    year_emb = _dot_t(onehot, ytab_ref[...])

    gvf = gv_ref[...].astype(jnp.float32)
    genre_emb = jax.nn.relu(_dot(gvf, wg_ref[...]) + bg_ref[...])
    t = jax.nn.relu(_dot_t(title_ref[...], wt1_ref[...]) + bt1_ref[...])
    text_emb = _dot(t, wt2_ref[...]) + bt2_ref[...]

    x = (_dot(item_emb, wb0_ref[0:64, :])
         + _dot(genre_emb, wb0_ref[64:96, :])
         + _dot(year_emb, wb0_ref[96:112, :])
         + _dot(text_emb, wb0_ref[112:208, :])
         + bb0_ref[...])
    x = _ln(jax.nn.relu(x), g0_ref[...], be0_ref[...])
    x = _ln(jax.nn.relu(_dot(x, wb1_ref[...]) + bb1_ref[...]), g1_ref[...], be1_ref[...])
    x = _ln(jax.nn.relu(_dot(x, wb2_ref[...]) + bb2_ref[...]), g2_ref[...], be2_ref[...])

    logits = _dot_t(x, wattn_ref[...]) + battn_ref[...]
    z = logits - jnp.max(logits, axis=-1, keepdims=True)
    e = jnp.exp(z)
    gw = e / jnp.sum(e, axis=-1, keepdims=True)
    w = gw * gvf * (gvf > 0.0).astype(jnp.float32)

    r1cat = jnp.concatenate([r1_ref[g] for g in range(NG)], axis=0)
    rb1cat = jnp.concatenate([rb1_ref[g:g + 1, :] for g in range(NG)], axis=1)
    h1 = jax.nn.relu(_dot_t(x, r1cat) + rb1cat)

    h2s = []
    for g in range(NG):
        h1g = h1[:, g * 64:(g + 1) * 64]
        h2s.append(jax.nn.relu(_dot_t(h1g, r2_ref[g]) + rb2_ref[g:g + 1, :]))
    h2 = jnp.concatenate(h2s, axis=1)
    lane = lax.broadcasted_iota(jnp.int32, (NG, NG * 32), 1)
    row = lax.broadcasted_iota(jnp.int32, (NG, NG * 32), 0)
    expand = (lane // 32 == row).astype(jnp.float32)
    wexp = _dot(w, expand)
    r3cat = jnp.concatenate([r3_ref[g] for g in range(NG)], axis=0)
    refin = _dot(h2 * wexp, r3cat) + _dot(w, rb3_ref[...])

    refined = jax.nn.relu(_dot(x, wagg_ref[0:128, :]) + _dot(refin, wagg_ref[128:160, :])
                          + bagg_ref[...])
    out = _ln(jax.nn.relu(_dot(refined, wo_ref[...]) + bo_ref[...]),
              go_ref[...], beo_ref[...])
    out_ref[...] = out


def _tc_specs(bt):
    def data(d):
        return pl.BlockSpec((bt, d), lambda i: (i, 0))

    def w1(n):
        return pl.BlockSpec((n,), lambda i: (0,))

    def w2(s):
        return pl.BlockSpec(s, lambda i: (0, 0))

    def w3(s):
        return pl.BlockSpec(s, lambda i: (0, 0, 0))

    in_specs = [
        pl.BlockSpec((bt, 2 * D_ITEM), lambda i: (i, 0)),
        data(1), data(1), data(NG), data(384),
        w2((16, YSPAN + 1)),
        w2((NG, 32)), w1(32),
        w2((192, 384)), w1(192),
        w2((192, 96)), w1(96),
        w2((208, 384)), w1(384), w1(384), w1(384),
        w2((384, 256)), w1(256), w1(256), w1(256),
        w2((256, 128)), w1(128), w1(128), w1(128),
        w2((NG, 128)), w1(NG),
        w3((NG, 64, 128)), w2((NG, 64)),
        w3((NG, 32, 64)), w2((NG, 32)),
        w3((NG, 32, 32)), w2((NG, 32)),
        w2((160, 128)), w1(128),
        w2((128, 128)), w1(128), w1(128), w1(128),
    ]
    out_spec = pl.BlockSpec((bt, 128), lambda i: (i, 0))
    return in_specs, out_spec


def _tc_args(slabs, item_ids, release_years, genre_vectors, title_embeddings, p):
    return (
        slabs, item_ids.reshape(B, 1), release_years.reshape(B, 1),
        genre_vectors, title_embeddings,
        p['year_table'].T,
        p['Wg'], p['bg'], p['Wt1'].T, p['bt1'], p['Wt2'], p['bt2'],
        p['Wb0'], p['bb0'], p['g0'], p['be0'],
        p['Wb1'], p['bb1'], p['g1'], p['be1'],
        p['Wb2'], p['bb2'], p['g2'], p['be2'],
        p['Wattn'].T, p['battn'],
        jnp.swapaxes(p['R1'], 1, 2), p['Rb1'],
        jnp.swapaxes(p['R2'], 1, 2), p['Rb2'], p['R3'], p['Rb3'],
        p['Wagg'], p['bagg'],
        p['Wo'], p['bo'], p['go'], p['beo'],
    )


def _tc_forward(slabs, item_ids, release_years, genre_vectors,
                title_embeddings, p, bt=1024):
    in_specs, out_spec = _tc_specs(bt)
    return pl.pallas_call(
        _tc_body,
        grid=(B // bt,),
        in_specs=in_specs,
        out_specs=out_spec,
        out_shape=jax.ShapeDtypeStruct((B, 128), jnp.float32),
    )(*_tc_args(slabs, item_ids, release_years, genre_vectors,
                title_embeddings, p))


def kernel(item_ids, genre_vectors, release_years, title_embeddings, params):
    itab2 = _tc_transpose(params['item_table'].T)
    slabs = _sc_gather(item_ids, itab2)
    return _tc_forward(slabs, item_ids, release_years, genre_vectors,
                       title_embeddings, params)


# speedup vs baseline: 1.4840x; 1.1040x over previous
import functools

import jax
import jax.numpy as jnp
from jax import lax
from jax.experimental import pallas as pl
from jax.experimental.pallas import tpu as pltpu
from jax.experimental.pallas import tpu_sc as plsc

B = 4096
NG = 18
YEAR_LO, YEAR_HI = 1919, 2000
YSPAN = YEAR_HI - YEAR_LO + 1
D_ITEM = 64


VOCAB = 100000
KIN = 4096
NPAIR = ((VOCAB + KIN - 1) // KIN) * (KIN // 2)


def _tc_transpose(item_table_t):
    def body(in_ref, out_ref):
        g = pl.program_id(0)
        lane = g * KIN + lax.broadcasted_iota(jnp.int32, (D_ITEM, KIN), 1)
        x = jnp.where(lane < VOCAB, in_ref[...], 0.0)
        eye = (lax.broadcasted_iota(jnp.int32, (D_ITEM, D_ITEM), 0)
               == lax.broadcasted_iota(jnp.int32, (D_ITEM, D_ITEM), 1)
               ).astype(jnp.float32)
        t = lax.dot_general(x, eye, (((0,), (0,)), ((), ())),
                            preferred_element_type=jnp.float32)
        for k in range(KIN // 256):
            out_ref[k * 128:(k + 1) * 128, 0:D_ITEM] = t[2 * k * 128:(2 * k + 1) * 128, :]
            out_ref[k * 128:(k + 1) * 128, D_ITEM:] = t[(2 * k + 1) * 128:(2 * k + 2) * 128, :]

    return pl.pallas_call(
        body,
        grid=(NPAIR * 2 // KIN,),
        in_specs=[pl.BlockSpec((D_ITEM, KIN), lambda g: (0, g))],
        out_specs=pl.BlockSpec((KIN // 2, 2 * D_ITEM), lambda g: (g, 0)),
        out_shape=jax.ShapeDtypeStruct((NPAIR, 2 * D_ITEM), jnp.float32),
    )(item_table_t)


def _sc_gather(item_ids, itab2):
    info = plsc.get_sparse_core_info()
    nw = info.num_cores * info.num_subcores
    bpw = B // nw

    mesh = plsc.VectorSubcoreMesh(core_axis_name="c", subcore_axis_name="s")

    @functools.partial(
        pl.kernel,
        mesh=mesh,
        compiler_params=pltpu.CompilerParams(use_tc_tiling_on_sc=True),
        out_type=jax.ShapeDtypeStruct((B, 2 * D_ITEM), jnp.float32),
        scratch_types=[
            pltpu.VMEM((bpw,), jnp.int32),
            pltpu.VMEM((bpw, 2 * D_ITEM), jnp.float32),
            pltpu.SemaphoreType.DMA,
        ],
    )
    def gather_kernel(ids_hbm, itab_hbm, out_hbm, idx_v, staged_v, sem):
        wid = lax.axis_index("s") * info.num_cores + lax.axis_index("c")
        base = wid * bpw
        pltpu.sync_copy(ids_hbm.at[pl.ds(base, bpw)], idx_v)
        for i in range(bpw // 16):
            v = idx_v[pl.ds(i * 16, 16)]
            j = (jax.lax.shift_right_logical(v, 8) * 128
                 + jnp.bitwise_and(v, 127))
            idx_v[pl.ds(i * 16, 16)] = j
        pltpu.async_copy(itab_hbm.at[idx_v], staged_v, sem).wait()
        pltpu.sync_copy(staged_v, out_hbm.at[pl.ds(base, bpw)])

    return gather_kernel(item_ids, itab2)


def _ln(x, g, b):
    m = jnp.mean(x, axis=-1, keepdims=True)
    q = jnp.mean(x * x, axis=-1, keepdims=True)
    v = q - m * m
    return (x - m) * lax.rsqrt(v + 1e-5) * g + b


def _dot(a, b):
    return jnp.dot(a, b, preferred_element_type=jnp.float32)


def _dot_t(a, bt):
    return lax.dot_general(a.astype(jnp.bfloat16), bt.astype(jnp.bfloat16),
                           (((1,), (1,)), ((), ())),
                           preferred_element_type=jnp.float32)


def _tc_body(slab_ref, ids_ref, yrs_ref, gv_ref, title_ref,
             ytab_ref,
             wg_ref, bg_ref, wt1_ref, bt1_ref, wt2_ref, bt2_ref,
             wb0_ref, bb0_ref, g0_ref, be0_ref,
             wb1_ref, bb1_ref, g1_ref, be1_ref,
             wb2_ref, bb2_ref, g2_ref, be2_ref,
             wattn_ref, battn_ref,
             r1_ref, rb1_ref, r2_ref, rb2_ref, r3_ref, rb3_ref,
             wagg_ref, bagg_ref, wo_ref, bo_ref, go_ref, beo_ref,
             out_ref):
    m = jnp.bitwise_and(jax.lax.shift_right_logical(ids_ref[...], 7),
                        1).astype(jnp.float32)
    item_emb = slab_ref[:, 0:D_ITEM] * (1.0 - m) + slab_ref[:, D_ITEM:2 * D_ITEM] * m

    yi = jnp.clip(yrs_ref[...] - YEAR_LO, 0, YSPAN - 1)
    onehot = (lax.broadcasted_iota(jnp.int32, (yi.shape[0], YSPAN + 1), 1)
              == yi).astype(jnp.float32)
    year_emb = _dot_t(onehot, ytab_ref[...])

    gvt = gv_ref[...].astype(jnp.float32)
    genre_emb = jax.nn.relu(
        lax.dot_general(gvt, wg_ref[...], (((0,), (0,)), ((), ())),
                        preferred_element_type=jnp.float32) + bg_ref[...])
    eye18 = (lax.broadcasted_iota(jnp.int32, (NG, NG), 0)
             == lax.broadcasted_iota(jnp.int32, (NG, NG), 1)).astype(jnp.float32)
    gvf = lax.dot_general(gvt, eye18, (((0,), (0,)), ((), ())),
                          preferred_element_type=jnp.float32)
    t = jax.nn.relu(_dot_t(title_ref[...], wt1_ref[...]) + bt1_ref[...])
    text_emb = _dot(t, wt2_ref[...]) + bt2_ref[...]

    x = (_dot(item_emb, wb0_ref[0:64, :])
         + _dot(genre_emb, wb0_ref[64:96, :])
         + _dot(year_emb, wb0_ref[96:112, :])
         + _dot(text_emb, wb0_ref[112:208, :])
         + bb0_ref[...])
    x = _ln(jax.nn.relu(x), g0_ref[...], be0_ref[...])
    x = _ln(jax.nn.relu(_dot(x, wb1_ref[...]) + bb1_ref[...]), g1_ref[...], be1_ref[...])
    x = _ln(jax.nn.relu(_dot(x, wb2_ref[...]) + bb2_ref[...]), g2_ref[...], be2_ref[...])

    logits = _dot_t(x, wattn_ref[...]) + battn_ref[...]
    z = logits - jnp.max(logits, axis=-1, keepdims=True)
    e = jnp.exp(z)
    gw = e / jnp.sum(e, axis=-1, keepdims=True)
    w = gw * gvf * (gvf > 0.0).astype(jnp.float32)

    r1cat = jnp.concatenate([r1_ref[g] for g in range(NG)], axis=0)
    rb1cat = jnp.concatenate([rb1_ref[g:g + 1, :] for g in range(NG)], axis=1)
    h1 = jax.nn.relu(_dot_t(x, r1cat) + rb1cat)

    h2s = []
    for g in range(NG):
        h1g = h1[:, g * 64:(g + 1) * 64]
        h2s.append(jax.nn.relu(_dot_t(h1g, r2_ref[g]) + rb2_ref[g:g + 1, :]))
    h2 = jnp.concatenate(h2s, axis=1)
    lane = lax.broadcasted_iota(jnp.int32, (NG, NG * 32), 1)
    row = lax.broadcasted_iota(jnp.int32, (NG, NG * 32), 0)
    expand = (lane // 32 == row).astype(jnp.float32)
    wexp = _dot(w, expand)
    r3cat = jnp.concatenate([r3_ref[g] for g in range(NG)], axis=0)
    refin = _dot(h2 * wexp, r3cat) + _dot(w, rb3_ref[...])

    refined = jax.nn.relu(_dot(x, wagg_ref[0:128, :]) + _dot(refin, wagg_ref[128:160, :])
                          + bagg_ref[...])
    out = _ln(jax.nn.relu(_dot(refined, wo_ref[...]) + bo_ref[...]),
              go_ref[...], beo_ref[...])
    out_ref[...] = out


def _tc_specs(bt):
    def data(d):
        return pl.BlockSpec((bt, d), lambda i: (i, 0))

    def w1(n):
        return pl.BlockSpec((n,), lambda i: (0,))

    def w2(s):
        return pl.BlockSpec(s, lambda i: (0, 0))

    def w3(s):
        return pl.BlockSpec(s, lambda i: (0, 0, 0))

    in_specs = [
        pl.BlockSpec((bt, 2 * D_ITEM), lambda i: (i, 0)),
        data(1), data(1),
        pl.BlockSpec((NG, bt), lambda i: (0, i)),
        data(384),
        w2((16, YSPAN + 1)),
        w2((NG, 32)), w1(32),
        w2((192, 384)), w1(192),
        w2((192, 96)), w1(96),
        w2((208, 384)), w1(384), w1(384), w1(384),
        w2((384, 256)), w1(256), w1(256), w1(256),
        w2((256, 128)), w1(128), w1(128), w1(128),
        w2((NG, 128)), w1(NG),
        w3((NG, 64, 128)), w2((NG, 64)),
        w3((NG, 32, 64)), w2((NG, 32)),
        w3((NG, 32, 32)), w2((NG, 32)),
        w2((160, 128)), w1(128),
        w2((128, 128)), w1(128), w1(128), w1(128),
    ]
    out_spec = pl.BlockSpec((bt, 128), lambda i: (i, 0))
    return in_specs, out_spec


def _tc_args(slabs, item_ids, release_years, genre_vectors, title_embeddings, p):
    return (
        slabs, item_ids.reshape(B, 1), release_years.reshape(B, 1),
        genre_vectors.T, title_embeddings,
        p['year_table'].T,
        p['Wg'], p['bg'], p['Wt1'].T, p['bt1'], p['Wt2'], p['bt2'],
        p['Wb0'], p['bb0'], p['g0'], p['be0'],
        p['Wb1'], p['bb1'], p['g1'], p['be1'],
        p['Wb2'], p['bb2'], p['g2'], p['be2'],
        p['Wattn'].T, p['battn'],
        jnp.swapaxes(p['R1'], 1, 2), p['Rb1'],
        jnp.swapaxes(p['R2'], 1, 2), p['Rb2'], p['R3'], p['Rb3'],
        p['Wagg'], p['bagg'],
        p['Wo'], p['bo'], p['go'], p['beo'],
    )


def _tc_forward(slabs, item_ids, release_years, genre_vectors,
                title_embeddings, p, bt=1024):
    in_specs, out_spec = _tc_specs(bt)
    return pl.pallas_call(
        _tc_body,
        grid=(B // bt,),
        in_specs=in_specs,
        out_specs=out_spec,
        out_shape=jax.ShapeDtypeStruct((B, 128), jnp.float32),
    )(*_tc_args(slabs, item_ids, release_years, genre_vectors,
                title_embeddings, p))


def kernel(item_ids, genre_vectors, release_years, title_embeddings, params):
    itab2 = _tc_transpose(params['item_table'].T)
    slabs = _sc_gather(item_ids, itab2)
    return _tc_forward(slabs, item_ids, release_years, genre_vectors,
                       title_embeddings, params)


# speedup vs baseline: 1.4964x; 1.0083x over previous
import functools

import jax
import jax.numpy as jnp
from jax import lax
from jax.experimental import pallas as pl
from jax.experimental.pallas import tpu as pltpu
from jax.experimental.pallas import tpu_sc as plsc

B = 4096
NG = 18
YEAR_LO, YEAR_HI = 1919, 2000
YSPAN = YEAR_HI - YEAR_LO + 1
D_ITEM = 64


VOCAB = 100000
KIN = 4096
NPAIR = ((VOCAB + KIN - 1) // KIN) * (KIN // 2)


def _tc_transpose(item_table_t):
    def body(in_ref, out_ref):
        g = pl.program_id(0)
        lane = g * KIN + lax.broadcasted_iota(jnp.int32, (D_ITEM, KIN), 1)
        x = jnp.where(lane < VOCAB, in_ref[...], 0.0)
        eye = (lax.broadcasted_iota(jnp.int32, (D_ITEM, D_ITEM), 0)
               == lax.broadcasted_iota(jnp.int32, (D_ITEM, D_ITEM), 1)
               ).astype(jnp.float32)
        t = lax.dot_general(x, eye, (((0,), (0,)), ((), ())),
                            preferred_element_type=jnp.float32)
        for k in range(KIN // 256):
            out_ref[k * 128:(k + 1) * 128, 0:D_ITEM] = t[2 * k * 128:(2 * k + 1) * 128, :]
            out_ref[k * 128:(k + 1) * 128, D_ITEM:] = t[(2 * k + 1) * 128:(2 * k + 2) * 128, :]

    return pl.pallas_call(
        body,
        grid=(NPAIR * 2 // KIN,),
        in_specs=[pl.BlockSpec((D_ITEM, KIN), lambda g: (0, g))],
        out_specs=pl.BlockSpec((KIN // 2, 2 * D_ITEM), lambda g: (g, 0)),
        out_shape=jax.ShapeDtypeStruct((NPAIR, 2 * D_ITEM), jnp.float32),
    )(item_table_t)


def _sc_gather(item_ids, itab2):
    info = plsc.get_sparse_core_info()
    nw = info.num_cores * info.num_subcores
    bpw = B // nw

    mesh = plsc.VectorSubcoreMesh(core_axis_name="c", subcore_axis_name="s")

    @functools.partial(
        pl.kernel,
        mesh=mesh,
        compiler_params=pltpu.CompilerParams(use_tc_tiling_on_sc=True),
        out_type=jax.ShapeDtypeStruct((B, 2 * D_ITEM), jnp.float32),
        scratch_types=[
            pltpu.VMEM((bpw,), jnp.int32),
            pltpu.VMEM((bpw, 2 * D_ITEM), jnp.float32),
            pltpu.SemaphoreType.DMA,
        ],
    )
    def gather_kernel(ids_hbm, itab_hbm, out_hbm, idx_v, staged_v, sem):
        wid = lax.axis_index("s") * info.num_cores + lax.axis_index("c")
        base = wid * bpw
        pltpu.sync_copy(ids_hbm.at[pl.ds(base, bpw)], idx_v)
        for i in range(bpw // 16):
            v = idx_v[pl.ds(i * 16, 16)]
            j = (jax.lax.shift_right_logical(v, 8) * 128
                 + jnp.bitwise_and(v, 127))
            idx_v[pl.ds(i * 16, 16)] = j
        pltpu.async_copy(itab_hbm.at[idx_v], staged_v, sem).wait()
        pltpu.sync_copy(staged_v, out_hbm.at[pl.ds(base, bpw)])

    return gather_kernel(item_ids, itab2)


def _ln(x, g, b):
    m = jnp.mean(x, axis=-1, keepdims=True)
    q = jnp.mean(x * x, axis=-1, keepdims=True)
    v = q - m * m
    return (x - m) * lax.rsqrt(v + 1e-5) * g + b


def _dot(a, b):
    return jnp.dot(a.astype(jnp.bfloat16), b.astype(jnp.bfloat16),
                   preferred_element_type=jnp.float32)


def _dot_t(a, bt):
    return lax.dot_general(a.astype(jnp.bfloat16), bt.astype(jnp.bfloat16),
                           (((1,), (1,)), ((), ())),
                           preferred_element_type=jnp.float32)


def _tc_body(slab_ref, ids_ref, yrs_ref, gv_ref, title_ref,
             ytab_ref,
             wg_ref, bg_ref, wt1_ref, bt1_ref, wt2_ref, bt2_ref,
             wb0_ref, bb0_ref, g0_ref, be0_ref,
             wb1_ref, bb1_ref, g1_ref, be1_ref,
             wb2_ref, bb2_ref, g2_ref, be2_ref,
             wattn_ref, battn_ref,
             r1_ref, rb1_ref, r2_ref, rb2_ref, r3_ref, rb3_ref,
             wagg_ref, bagg_ref, wo_ref, bo_ref, go_ref, beo_ref,
             out_ref):
    m = jnp.bitwise_and(jax.lax.shift_right_logical(ids_ref[...], 7),
                        1).astype(jnp.float32)
    item_emb = slab_ref[:, 0:D_ITEM] * (1.0 - m) + slab_ref[:, D_ITEM:2 * D_ITEM] * m

    yi = jnp.clip(yrs_ref[...] - YEAR_LO, 0, YSPAN - 1)
    onehot = (lax.broadcasted_iota(jnp.int32, (yi.shape[0], YSPAN + 1), 1)
              == yi).astype(jnp.float32)
    year_emb = _dot_t(onehot, ytab_ref[...])

    gvt = gv_ref[...].astype(jnp.float32)
    genre_emb = jax.nn.relu(
        lax.dot_general(gvt, wg_ref[...], (((0,), (0,)), ((), ())),
                        preferred_element_type=jnp.float32) + bg_ref[...])
    eye18 = (lax.broadcasted_iota(jnp.int32, (NG, NG), 0)
             == lax.broadcasted_iota(jnp.int32, (NG, NG), 1)).astype(jnp.float32)
    gvf = lax.dot_general(gvt, eye18, (((0,), (0,)), ((), ())),
                          preferred_element_type=jnp.float32)
    t = jax.nn.relu(_dot_t(title_ref[...], wt1_ref[...]) + bt1_ref[...])
    text_emb = _dot(t, wt2_ref[...]) + bt2_ref[...]

    x = (_dot(item_emb, wb0_ref[0:64, :])
         + _dot(genre_emb, wb0_ref[64:96, :])
         + _dot(year_emb, wb0_ref[96:112, :])
         + _dot(text_emb, wb0_ref[112:208, :])
         + bb0_ref[...])
    x = _ln(jax.nn.relu(x), g0_ref[...], be0_ref[...])
    x = _ln(jax.nn.relu(_dot(x, wb1_ref[...]) + bb1_ref[...]), g1_ref[...], be1_ref[...])
    x = _ln(jax.nn.relu(_dot(x, wb2_ref[...]) + bb2_ref[...]), g2_ref[...], be2_ref[...])

    logits = _dot_t(x, wattn_ref[...]) + battn_ref[...]
    z = logits - jnp.max(logits, axis=-1, keepdims=True)
    e = jnp.exp(z)
    gw = e / jnp.sum(e, axis=-1, keepdims=True)
    w = gw * gvf * (gvf > 0.0).astype(jnp.float32)

    r1cat = jnp.concatenate([r1_ref[g] for g in range(NG)], axis=0)
    rb1cat = jnp.concatenate([rb1_ref[g:g + 1, :] for g in range(NG)], axis=1)
    h1 = jax.nn.relu(_dot_t(x, r1cat) + rb1cat)

    h2s = []
    for g in range(NG):
        h1g = h1[:, g * 64:(g + 1) * 64]
        h2s.append(jax.nn.relu(_dot_t(h1g, r2_ref[g]) + rb2_ref[g:g + 1, :]))
    h2 = jnp.concatenate(h2s, axis=1)
    lane = lax.broadcasted_iota(jnp.int32, (NG, NG * 32), 1)
    row = lax.broadcasted_iota(jnp.int32, (NG, NG * 32), 0)
    expand = (lane // 32 == row).astype(jnp.float32)
    wexp = _dot(w, expand)
    r3cat = jnp.concatenate([r3_ref[g] for g in range(NG)], axis=0)
    refin = _dot(h2 * wexp, r3cat) + _dot(w, rb3_ref[...])

    refined = jax.nn.relu(_dot(x, wagg_ref[0:128, :]) + _dot(refin, wagg_ref[128:160, :])
                          + bagg_ref[...])
    out = _ln(jax.nn.relu(_dot(refined, wo_ref[...]) + bo_ref[...]),
              go_ref[...], beo_ref[...])
    out_ref[...] = out


def _tc_specs(bt):
    def data(d):
        return pl.BlockSpec((bt, d), lambda i: (i, 0))

    def w1(n):
        return pl.BlockSpec((n,), lambda i: (0,))

    def w2(s):
        return pl.BlockSpec(s, lambda i: (0, 0))

    def w3(s):
        return pl.BlockSpec(s, lambda i: (0, 0, 0))

    in_specs = [
        pl.BlockSpec((bt, 2 * D_ITEM), lambda i: (i, 0)),
        data(1), data(1),
        pl.BlockSpec((NG, bt), lambda i: (0, i)),
        data(384),
        w2((16, YSPAN + 1)),
        w2((NG, 32)), w1(32),
        w2((192, 384)), w1(192),
        w2((192, 96)), w1(96),
        w2((208, 384)), w1(384), w1(384), w1(384),
        w2((384, 256)), w1(256), w1(256), w1(256),
        w2((256, 128)), w1(128), w1(128), w1(128),
        w2((NG, 128)), w1(NG),
        w3((NG, 64, 128)), w2((NG, 64)),
        w3((NG, 32, 64)), w2((NG, 32)),
        w3((NG, 32, 32)), w2((NG, 32)),
        w2((160, 128)), w1(128),
        w2((128, 128)), w1(128), w1(128), w1(128),
    ]
    out_spec = pl.BlockSpec((bt, 128), lambda i: (i, 0))
    return in_specs, out_spec


def _tc_args(slabs, item_ids, release_years, genre_vectors, title_embeddings, p):
    return (
        slabs, item_ids.reshape(B, 1), release_years.reshape(B, 1),
        genre_vectors.T, title_embeddings,
        p['year_table'].T,
        p['Wg'], p['bg'], p['Wt1'].T, p['bt1'], p['Wt2'], p['bt2'],
        p['Wb0'], p['bb0'], p['g0'], p['be0'],
        p['Wb1'], p['bb1'], p['g1'], p['be1'],
        p['Wb2'], p['bb2'], p['g2'], p['be2'],
        p['Wattn'].T, p['battn'],
        jnp.swapaxes(p['R1'], 1, 2), p['Rb1'],
        jnp.swapaxes(p['R2'], 1, 2), p['Rb2'], p['R3'], p['Rb3'],
        p['Wagg'], p['bagg'],
        p['Wo'], p['bo'], p['go'], p['beo'],
    )


def _tc_forward(slabs, item_ids, release_years, genre_vectors,
                title_embeddings, p, bt=1024):
    in_specs, out_spec = _tc_specs(bt)
    return pl.pallas_call(
        _tc_body,
        grid=(B // bt,),
        in_specs=in_specs,
        out_specs=out_spec,
        out_shape=jax.ShapeDtypeStruct((B, 128), jnp.float32),
    )(*_tc_args(slabs, item_ids, release_years, genre_vectors,
                title_embeddings, p))


def kernel(item_ids, genre_vectors, release_years, title_embeddings, params):
    itab2 = _tc_transpose(params['item_table'].T)
    slabs = _sc_gather(item_ids, itab2)
    return _tc_forward(slabs, item_ids, release_years, genre_vectors,
                       title_embeddings, params)


# speedup vs baseline: 1.6653x; 1.1129x over previous
import functools

import jax
import jax.numpy as jnp
from jax import lax
from jax.experimental import pallas as pl
from jax.experimental.pallas import tpu as pltpu
from jax.experimental.pallas import tpu_sc as plsc

B = 4096
NG = 18
YEAR_LO, YEAR_HI = 1919, 2000
YSPAN = YEAR_HI - YEAR_LO + 1
D_ITEM = 64


VOCAB = 100000
KIN = 8192
NPAIR = ((VOCAB + KIN - 1) // KIN) * (KIN // 2)


def _tc_transpose(item_table_t):
    def body(in_ref, out_ref):
        g = pl.program_id(0)
        lane = g * KIN + lax.broadcasted_iota(jnp.int32, (D_ITEM, KIN), 1)
        x = jnp.where(lane < VOCAB, in_ref[...], 0.0)
        eye = (lax.broadcasted_iota(jnp.int32, (D_ITEM, D_ITEM), 0)
               == lax.broadcasted_iota(jnp.int32, (D_ITEM, D_ITEM), 1)
               ).astype(jnp.float32)
        t = lax.dot_general(x, eye, (((0,), (0,)), ((), ())),
                            preferred_element_type=jnp.float32)
        for k in range(KIN // 256):
            out_ref[k * 128:(k + 1) * 128, 0:D_ITEM] = t[2 * k * 128:(2 * k + 1) * 128, :]
            out_ref[k * 128:(k + 1) * 128, D_ITEM:] = t[(2 * k + 1) * 128:(2 * k + 2) * 128, :]

    return pl.pallas_call(
        body,
        grid=(NPAIR * 2 // KIN,),
        in_specs=[pl.BlockSpec((D_ITEM, KIN), lambda g: (0, g))],
        out_specs=pl.BlockSpec((KIN // 2, 2 * D_ITEM), lambda g: (g, 0)),
        out_shape=jax.ShapeDtypeStruct((NPAIR, 2 * D_ITEM), jnp.float32),
    )(item_table_t)


def _sc_gather(item_ids, itab2):
    info = plsc.get_sparse_core_info()
    nw = info.num_cores * info.num_subcores
    bpw = B // nw

    mesh = plsc.VectorSubcoreMesh(core_axis_name="c", subcore_axis_name="s")

    @functools.partial(
        pl.kernel,
        mesh=mesh,
        compiler_params=pltpu.CompilerParams(use_tc_tiling_on_sc=True),
        out_type=jax.ShapeDtypeStruct((B, 2 * D_ITEM), jnp.float32),
        scratch_types=[
            pltpu.VMEM((bpw,), jnp.int32),
            pltpu.VMEM((bpw, 2 * D_ITEM), jnp.float32),
            pltpu.SemaphoreType.DMA,
        ],
    )
    def gather_kernel(ids_hbm, itab_hbm, out_hbm, idx_v, staged_v, sem):
        wid = lax.axis_index("s") * info.num_cores + lax.axis_index("c")
        base = wid * bpw
        pltpu.sync_copy(ids_hbm.at[pl.ds(base, bpw)], idx_v)
        for i in range(bpw // 16):
            v = idx_v[pl.ds(i * 16, 16)]
            j = (jax.lax.shift_right_logical(v, 8) * 128
                 + jnp.bitwise_and(v, 127))
            idx_v[pl.ds(i * 16, 16)] = j
        pltpu.async_copy(itab_hbm.at[idx_v], staged_v, sem).wait()
        pltpu.sync_copy(staged_v, out_hbm.at[pl.ds(base, bpw)])

    return gather_kernel(item_ids, itab2)


def _ln(x, g, b):
    m = jnp.mean(x, axis=-1, keepdims=True)
    q = jnp.mean(x * x, axis=-1, keepdims=True)
    v = q - m * m
    return (x - m) * lax.rsqrt(v + 1e-5) * g + b


def _dot(a, b):
    return jnp.dot(a.astype(jnp.bfloat16), b.astype(jnp.bfloat16),
                   preferred_element_type=jnp.float32)


def _dot_t(a, bt):
    return lax.dot_general(a.astype(jnp.bfloat16), bt.astype(jnp.bfloat16),
                           (((1,), (1,)), ((), ())),
                           preferred_element_type=jnp.float32)


def _tc_body(slab_ref, ids_ref, yrs_ref, gv_ref, title_ref,
             ytab_ref,
             wg_ref, bg_ref, wt1_ref, bt1_ref, wt2_ref, bt2_ref,
             wb0_ref, bb0_ref, g0_ref, be0_ref,
             wb1_ref, bb1_ref, g1_ref, be1_ref,
             wb2_ref, bb2_ref, g2_ref, be2_ref,
             wattn_ref, battn_ref,
             r1_ref, rb1_ref, r2_ref, rb2_ref, r3_ref, rb3_ref,
             wagg_ref, bagg_ref, wo_ref, bo_ref, go_ref, beo_ref,
             out_ref):
    m = jnp.bitwise_and(jax.lax.shift_right_logical(ids_ref[...], 7),
                        1).astype(jnp.float32)
    item_emb = slab_ref[:, 0:D_ITEM] * (1.0 - m) + slab_ref[:, D_ITEM:2 * D_ITEM] * m

    yi = jnp.clip(yrs_ref[...] - YEAR_LO, 0, YSPAN - 1)
    onehot = (lax.broadcasted_iota(jnp.int32, (yi.shape[0], YSPAN + 1), 1)
              == yi).astype(jnp.float32)
    year_emb = _dot_t(onehot, ytab_ref[...])

    gvt = gv_ref[...].astype(jnp.float32)
    genre_emb = jax.nn.relu(
        lax.dot_general(gvt, wg_ref[...], (((0,), (0,)), ((), ())),
                        preferred_element_type=jnp.float32) + bg_ref[...])
    eye18 = (lax.broadcasted_iota(jnp.int32, (NG, NG), 0)
             == lax.broadcasted_iota(jnp.int32, (NG, NG), 1)).astype(jnp.float32)
    gvf = lax.dot_general(gvt, eye18, (((0,), (0,)), ((), ())),
                          preferred_element_type=jnp.float32)
    t = jax.nn.relu(_dot_t(title_ref[...], wt1_ref[...]) + bt1_ref[...])
    text_emb = _dot(t, wt2_ref[...]) + bt2_ref[...]

    x = (_dot(item_emb, wb0_ref[0:64, :])
         + _dot(genre_emb, wb0_ref[64:96, :])
         + _dot(year_emb, wb0_ref[96:112, :])
         + _dot(text_emb, wb0_ref[112:208, :])
         + bb0_ref[...])
    x = _ln(jax.nn.relu(x), g0_ref[...], be0_ref[...])
    x = _ln(jax.nn.relu(_dot(x, wb1_ref[...]) + bb1_ref[...]), g1_ref[...], be1_ref[...])
    x = _ln(jax.nn.relu(_dot(x, wb2_ref[...]) + bb2_ref[...]), g2_ref[...], be2_ref[...])

    logits = _dot_t(x, wattn_ref[...]) + battn_ref[...]
    z = logits - jnp.max(logits, axis=-1, keepdims=True)
    e = jnp.exp(z)
    gw = e / jnp.sum(e, axis=-1, keepdims=True)
    w = gw * gvf * (gvf > 0.0).astype(jnp.float32)

    r1cat = jnp.concatenate([r1_ref[g] for g in range(NG)], axis=0)
    rb1cat = jnp.concatenate([rb1_ref[g:g + 1, :] for g in range(NG)], axis=1)
    h1 = jax.nn.relu(_dot_t(x, r1cat) + rb1cat)

    h2s = []
    for g in range(NG):
        h1g = h1[:, g * 64:(g + 1) * 64]
        h2s.append(jax.nn.relu(_dot_t(h1g, r2_ref[g]) + rb2_ref[g:g + 1, :]))
    h2 = jnp.concatenate(h2s, axis=1)
    lane = lax.broadcasted_iota(jnp.int32, (NG, NG * 32), 1)
    row = lax.broadcasted_iota(jnp.int32, (NG, NG * 32), 0)
    expand = (lane // 32 == row).astype(jnp.float32)
    wexp = _dot(w, expand)
    r3cat = jnp.concatenate([r3_ref[g] for g in range(NG)], axis=0)
    refin = _dot(h2 * wexp, r3cat) + _dot(w, rb3_ref[...])

    refined = jax.nn.relu(_dot(x, wagg_ref[0:128, :]) + _dot(refin, wagg_ref[128:160, :])
                          + bagg_ref[...])
    out = _ln(jax.nn.relu(_dot(refined, wo_ref[...]) + bo_ref[...]),
              go_ref[...], beo_ref[...])
    out_ref[...] = out


def _tc_specs(bt):
    def data(d):
        return pl.BlockSpec((bt, d), lambda i: (i, 0))

    def w1(n):
        return pl.BlockSpec((n,), lambda i: (0,))

    def w2(s):
        return pl.BlockSpec(s, lambda i: (0, 0))

    def w3(s):
        return pl.BlockSpec(s, lambda i: (0, 0, 0))

    in_specs = [
        pl.BlockSpec((bt, 2 * D_ITEM), lambda i: (i, 0)),
        data(1), data(1),
        pl.BlockSpec((NG, bt), lambda i: (0, i)),
        data(384),
        w2((16, YSPAN + 1)),
        w2((NG, 32)), w1(32),
        w2((192, 384)), w1(192),
        w2((192, 96)), w1(96),
        w2((208, 384)), w1(384), w1(384), w1(384),
        w2((384, 256)), w1(256), w1(256), w1(256),
        w2((256, 128)), w1(128), w1(128), w1(128),
        w2((NG, 128)), w1(NG),
        w3((NG, 64, 128)), w2((NG, 64)),
        w3((NG, 32, 64)), w2((NG, 32)),
        w3((NG, 32, 32)), w2((NG, 32)),
        w2((160, 128)), w1(128),
        w2((128, 128)), w1(128), w1(128), w1(128),
    ]
    out_spec = pl.BlockSpec((bt, 128), lambda i: (i, 0))
    return in_specs, out_spec


def _tc_args(slabs, item_ids, release_years, genre_vectors, title_embeddings, p):
    return (
        slabs, item_ids.reshape(B, 1), release_years.reshape(B, 1),
        genre_vectors.T, title_embeddings,
        p['year_table'].T,
        p['Wg'], p['bg'], p['Wt1'].T, p['bt1'], p['Wt2'], p['bt2'],
        p['Wb0'], p['bb0'], p['g0'], p['be0'],
        p['Wb1'], p['bb1'], p['g1'], p['be1'],
        p['Wb2'], p['bb2'], p['g2'], p['be2'],
        p['Wattn'].T, p['battn'],
        jnp.swapaxes(p['R1'], 1, 2), p['Rb1'],
        jnp.swapaxes(p['R2'], 1, 2), p['Rb2'], p['R3'], p['Rb3'],
        p['Wagg'], p['bagg'],
        p['Wo'], p['bo'], p['go'], p['beo'],
    )


def _tc_forward(slabs, item_ids, release_years, genre_vectors,
                title_embeddings, p, bt=2048):
    in_specs, out_spec = _tc_specs(bt)
    return pl.pallas_call(
        _tc_body,
        grid=(B // bt,),
        in_specs=in_specs,
        out_specs=out_spec,
        out_shape=jax.ShapeDtypeStruct((B, 128), jnp.float32),
    )(*_tc_args(slabs, item_ids, release_years, genre_vectors,
                title_embeddings, p))


def kernel(item_ids, genre_vectors, release_years, title_embeddings, params):
    itab2 = _tc_transpose(params['item_table'].T)
    slabs = _sc_gather(item_ids, itab2)
    return _tc_forward(slabs, item_ids, release_years, genre_vectors,
                       title_embeddings, params)


# speedup vs baseline: 1.6704x; 1.0031x over previous
import functools

import jax
import jax.numpy as jnp
from jax import lax
from jax.experimental import pallas as pl
from jax.experimental.pallas import tpu as pltpu
from jax.experimental.pallas import tpu_sc as plsc

B = 4096
NG = 18
YEAR_LO, YEAR_HI = 1919, 2000
YSPAN = YEAR_HI - YEAR_LO + 1
D_ITEM = 64


VOCAB = 100000
KIN = 16384
NPAIR = ((VOCAB + KIN - 1) // KIN) * (KIN // 2)


def _tc_transpose(item_table_t):
    def body(in_ref, out_ref):
        g = pl.program_id(0)
        lane = g * KIN + lax.broadcasted_iota(jnp.int32, (D_ITEM, KIN), 1)
        x = jnp.where(lane < VOCAB, in_ref[...], 0.0)
        eye = (lax.broadcasted_iota(jnp.int32, (D_ITEM, D_ITEM), 0)
               == lax.broadcasted_iota(jnp.int32, (D_ITEM, D_ITEM), 1)
               ).astype(jnp.float32)
        t = lax.dot_general(x, eye, (((0,), (0,)), ((), ())),
                            preferred_element_type=jnp.float32)
        for k in range(KIN // 256):
            out_ref[k * 128:(k + 1) * 128, 0:D_ITEM] = t[2 * k * 128:(2 * k + 1) * 128, :]
            out_ref[k * 128:(k + 1) * 128, D_ITEM:] = t[(2 * k + 1) * 128:(2 * k + 2) * 128, :]

    return pl.pallas_call(
        body,
        grid=(NPAIR * 2 // KIN,),
        in_specs=[pl.BlockSpec((D_ITEM, KIN), lambda g: (0, g))],
        out_specs=pl.BlockSpec((KIN // 2, 2 * D_ITEM), lambda g: (g, 0)),
        out_shape=jax.ShapeDtypeStruct((NPAIR, 2 * D_ITEM), jnp.float32),
    )(item_table_t)


def _sc_gather(item_ids, itab2):
    info = plsc.get_sparse_core_info()
    nw = info.num_cores * info.num_subcores
    bpw = B // nw

    mesh = plsc.VectorSubcoreMesh(core_axis_name="c", subcore_axis_name="s")

    @functools.partial(
        pl.kernel,
        mesh=mesh,
        compiler_params=pltpu.CompilerParams(use_tc_tiling_on_sc=True),
        out_type=jax.ShapeDtypeStruct((B, 2 * D_ITEM), jnp.float32),
        scratch_types=[
            pltpu.VMEM((bpw,), jnp.int32),
            pltpu.VMEM((bpw, 2 * D_ITEM), jnp.float32),
            pltpu.SemaphoreType.DMA,
        ],
    )
    def gather_kernel(ids_hbm, itab_hbm, out_hbm, idx_v, staged_v, sem):
        wid = lax.axis_index("s") * info.num_cores + lax.axis_index("c")
        base = wid * bpw
        pltpu.sync_copy(ids_hbm.at[pl.ds(base, bpw)], idx_v)
        for i in range(bpw // 16):
            v = idx_v[pl.ds(i * 16, 16)]
            j = (jax.lax.shift_right_logical(v, 8) * 128
                 + jnp.bitwise_and(v, 127))
            idx_v[pl.ds(i * 16, 16)] = j
        pltpu.async_copy(itab_hbm.at[idx_v], staged_v, sem).wait()
        pltpu.sync_copy(staged_v, out_hbm.at[pl.ds(base, bpw)])

    return gather_kernel(item_ids, itab2)


def _ln(x, g, b):
    m = jnp.mean(x, axis=-1, keepdims=True)
    q = jnp.mean(x * x, axis=-1, keepdims=True)
    v = q - m * m
    return (x - m) * lax.rsqrt(v + 1e-5) * g + b


def _dot(a, b):
    return jnp.dot(a.astype(jnp.bfloat16), b.astype(jnp.bfloat16),
                   preferred_element_type=jnp.float32)


def _dot_t(a, bt):
    return lax.dot_general(a.astype(jnp.bfloat16), bt.astype(jnp.bfloat16),
                           (((1,), (1,)), ((), ())),
                           preferred_element_type=jnp.float32)


def _tc_body(slab_ref, ids_ref, yrs_ref, gv_ref, title_ref,
             ytab_ref,
             wg_ref, bg_ref, wt1_ref, bt1_ref, wt2_ref, bt2_ref,
             wb0_ref, bb0_ref, g0_ref, be0_ref,
             wb1_ref, bb1_ref, g1_ref, be1_ref,
             wb2_ref, bb2_ref, g2_ref, be2_ref,
             wattn_ref, battn_ref,
             r1_ref, rb1_ref, r2_ref, rb2_ref, r3_ref, rb3_ref,
             wagg_ref, bagg_ref, wo_ref, bo_ref, go_ref, beo_ref,
             out_ref):
    m = jnp.bitwise_and(jax.lax.shift_right_logical(ids_ref[...], 7),
                        1).astype(jnp.float32)
    item_emb = slab_ref[:, 0:D_ITEM] * (1.0 - m) + slab_ref[:, D_ITEM:2 * D_ITEM] * m

    yi = jnp.clip(yrs_ref[...] - YEAR_LO, 0, YSPAN - 1)
    onehot = (lax.broadcasted_iota(jnp.int32, (yi.shape[0], YSPAN + 1), 1)
              == yi).astype(jnp.float32)
    year_emb = _dot_t(onehot, ytab_ref[...])

    gvt = gv_ref[...].astype(jnp.float32)
    genre_emb = jax.nn.relu(
        lax.dot_general(gvt, wg_ref[...], (((0,), (0,)), ((), ())),
                        preferred_element_type=jnp.float32) + bg_ref[...])
    eye18 = (lax.broadcasted_iota(jnp.int32, (NG, NG), 0)
             == lax.broadcasted_iota(jnp.int32, (NG, NG), 1)).astype(jnp.float32)
    gvf = lax.dot_general(gvt, eye18, (((0,), (0,)), ((), ())),
                          preferred_element_type=jnp.float32)
    t = jax.nn.relu(_dot_t(title_ref[...], wt1_ref[...]) + bt1_ref[...])
    text_emb = _dot(t, wt2_ref[...]) + bt2_ref[...]

    x = (_dot(item_emb, wb0_ref[0:64, :])
         + _dot(genre_emb, wb0_ref[64:96, :])
         + _dot(year_emb, wb0_ref[96:112, :])
         + _dot(text_emb, wb0_ref[112:208, :])
         + bb0_ref[...])
    x = _ln(jax.nn.relu(x), g0_ref[...], be0_ref[...])
    x = _ln(jax.nn.relu(_dot(x, wb1_ref[...]) + bb1_ref[...]), g1_ref[...], be1_ref[...])
    x = _ln(jax.nn.relu(_dot(x, wb2_ref[...]) + bb2_ref[...]), g2_ref[...], be2_ref[...])

    logits = _dot_t(x, wattn_ref[...]) + battn_ref[...]
    z = logits - jnp.max(logits, axis=-1, keepdims=True)
    e = jnp.exp(z)
    gw = e / jnp.sum(e, axis=-1, keepdims=True)
    w = gw * gvf * (gvf > 0.0).astype(jnp.float32)

    r1cat = jnp.concatenate([r1_ref[g] for g in range(NG)], axis=0)
    rb1cat = jnp.concatenate([rb1_ref[g:g + 1, :] for g in range(NG)], axis=1)
    h1 = jax.nn.relu(_dot_t(x, r1cat) + rb1cat)

    h2s = []
    for g in range(NG):
        h1g = h1[:, g * 64:(g + 1) * 64]
        h2s.append(jax.nn.relu(_dot_t(h1g, r2_ref[g]) + rb2_ref[g:g + 1, :]))
    h2 = jnp.concatenate(h2s, axis=1)
    lane = lax.broadcasted_iota(jnp.int32, (NG, NG * 32), 1)
    row = lax.broadcasted_iota(jnp.int32, (NG, NG * 32), 0)
    expand = (lane // 32 == row).astype(jnp.float32)
    wexp = _dot(w, expand)
    r3cat = jnp.concatenate([r3_ref[g] for g in range(NG)], axis=0)
    refin = _dot(h2 * wexp, r3cat) + _dot(w, rb3_ref[...])

    refined = jax.nn.relu(_dot(x, wagg_ref[0:128, :]) + _dot(refin, wagg_ref[128:160, :])
                          + bagg_ref[...])
    out = _ln(jax.nn.relu(_dot(refined, wo_ref[...]) + bo_ref[...]),
              go_ref[...], beo_ref[...])
    out_ref[...] = out


def _tc_specs(bt):
    def data(d):
        return pl.BlockSpec((bt, d), lambda i: (i, 0))

    def w1(n):
        return pl.BlockSpec((n,), lambda i: (0,))

    def w2(s):
        return pl.BlockSpec(s, lambda i: (0, 0))

    def w3(s):
        return pl.BlockSpec(s, lambda i: (0, 0, 0))

    in_specs = [
        pl.BlockSpec((bt, 2 * D_ITEM), lambda i: (i, 0)),
        data(1), data(1),
        pl.BlockSpec((NG, bt), lambda i: (0, i)),
        data(384),
        w2((16, YSPAN + 1)),
        w2((NG, 32)), w1(32),
        w2((192, 384)), w1(192),
        w2((192, 96)), w1(96),
        w2((208, 384)), w1(384), w1(384), w1(384),
        w2((384, 256)), w1(256), w1(256), w1(256),
        w2((256, 128)), w1(128), w1(128), w1(128),
        w2((NG, 128)), w1(NG),
        w3((NG, 64, 128)), w2((NG, 64)),
        w3((NG, 32, 64)), w2((NG, 32)),
        w3((NG, 32, 32)), w2((NG, 32)),
        w2((160, 128)), w1(128),
        w2((128, 128)), w1(128), w1(128), w1(128),
    ]
    out_spec = pl.BlockSpec((bt, 128), lambda i: (i, 0))
    return in_specs, out_spec


def _tc_args(slabs, item_ids, release_years, genre_vectors, title_embeddings, p):
    return (
        slabs, item_ids.reshape(B, 1), release_years.reshape(B, 1),
        genre_vectors.T, title_embeddings,
        p['year_table'].T,
        p['Wg'], p['bg'], p['Wt1'].T, p['bt1'], p['Wt2'], p['bt2'],
        p['Wb0'], p['bb0'], p['g0'], p['be0'],
        p['Wb1'], p['bb1'], p['g1'], p['be1'],
        p['Wb2'], p['bb2'], p['g2'], p['be2'],
        p['Wattn'].T, p['battn'],
        jnp.swapaxes(p['R1'], 1, 2), p['Rb1'],
        jnp.swapaxes(p['R2'], 1, 2), p['Rb2'], p['R3'], p['Rb3'],
        p['Wagg'], p['bagg'],
        p['Wo'], p['bo'], p['go'], p['beo'],
    )


def _tc_forward(slabs, item_ids, release_years, genre_vectors,
                title_embeddings, p, bt=2048):
    in_specs, out_spec = _tc_specs(bt)
    return pl.pallas_call(
        _tc_body,
        grid=(B // bt,),
        in_specs=in_specs,
        out_specs=out_spec,
        out_shape=jax.ShapeDtypeStruct((B, 128), jnp.float32),
    )(*_tc_args(slabs, item_ids, release_years, genre_vectors,
                title_embeddings, p))


def kernel(item_ids, genre_vectors, release_years, title_embeddings, params):
    itab2 = _tc_transpose(params['item_table'].T)
    slabs = _sc_gather(item_ids, itab2)
    return _tc_forward(slabs, item_ids, release_years, genre_vectors,
                       title_embeddings, params)
